# baseline (speedup 1.0000x reference)
import functools

import jax
import jax.numpy as jnp
from jax import lax
from jax.experimental import pallas as pl
from jax.experimental.pallas import tpu as pltpu
from jax.experimental.pallas import tpu_sc as plsc

NC = 2
NS = 16
NW = NC * NS
CH = 128
GARB = 240

_MESH = dict(core_axis_name="c", subcore_axis_name="s")

_SC_TILING = pltpu.CompilerParams(use_tc_tiling_on_sc=False)


def _zero_fill_1d(ref, size):
    def b(i, carry):
        ref[pl.ds(i * 16, 16)] = jnp.zeros((16,), jnp.float32)
        return carry

    lax.fori_loop(0, size // 16, b, 0)


def _zero_fill_2d(ref, r, d):
    def b(i, carry):
        for j in range(d // 16):
            ref[i, pl.ds(16 * j, 16)] = jnp.zeros((16,), jnp.float32)
        return carry

    lax.fori_loop(0, r, b, 0)


def _pad_edges(idx_val, idx_tgt, n):
    e = idx_val.shape[0]
    ep = ((e + NW * 8 - 1) // (NW * 8)) * (NW * 8)
    pad = ep - e
    if pad == 0:
        return idx_val, idx_tgt, ep
    ar = jnp.arange(pad, dtype=jnp.int32)
    val_p = jnp.concatenate([idx_val, ar % n])
    tgt_p = jnp.concatenate([idx_tgt, n + (ar % GARB)])
    return val_p, tgt_p, ep


def _make_deg(n_acc, ep):
    ew = ep // NW
    cpt = ew // CH
    tail = ew % CH
    rpt = n_acc // NS
    mesh = plsc.VectorSubcoreMesh(**_MESH)

    @functools.partial(
        pl.kernel,
        mesh=mesh,
        out_type=jax.ShapeDtypeStruct((NC * n_acc,), jnp.float32),
        scratch_types=[
            pltpu.VMEM((CH,), jnp.int32),
            pltpu.VMEM((CH,), jnp.int32),
            pltpu.VMEM((CH,), jnp.float32),
            pltpu.VMEM((rpt,), jnp.float32),
            pltpu.VMEM_SHARED((n_acc,), jnp.float32),
        ],
    )
    def deg_k(dst_hbm, out_hbm, idx_a, idx_b, ones_v, zbuf, deg_sh):
        c = lax.axis_index("c")
        s = lax.axis_index("s")
        wid = c * NS + s
        for j in range(CH // 16):
            ones_v[pl.ds(16 * j, 16)] = jnp.ones((16,), jnp.float32)
        if CH % 16:
            ones_v[pl.ds(CH - 16, 16)] = jnp.ones((16,), jnp.float32)
        r0 = s * rpt
        _zero_fill_1d(zbuf, rpt)
        pltpu.sync_copy(zbuf, deg_sh.at[pl.ds(r0, rpt)])
        plsc.subcore_barrier()

        def body(i, carry):
            base_a = wid * ew + (2 * i) * CH
            pltpu.sync_copy(dst_hbm.at[pl.ds(base_a, CH)], idx_a)
            pltpu.sync_copy(dst_hbm.at[pl.ds(base_a + CH, CH)], idx_b)
            pltpu.sync_copy(ones_v, deg_sh.at[idx_a], add=True)
            pltpu.sync_copy(ones_v, deg_sh.at[idx_b], add=True)
            return carry

        lax.fori_loop(0, cpt // 2, body, 0)
        if cpt % 2:
            base = wid * ew + (cpt - 1) * CH
            pltpu.sync_copy(dst_hbm.at[pl.ds(base, CH)], idx_a)
            pltpu.sync_copy(ones_v, deg_sh.at[idx_a], add=True)
        if tail:
            base = wid * ew + cpt * CH
            for j in range(CH // 16):
                garb = 16 * j + jnp.arange(16, dtype=jnp.int32)
                idx_a[pl.ds(16 * j, 16)] = (n_acc - GARB) + garb % GARB
            pltpu.sync_copy(dst_hbm.at[pl.ds(base, tail)],
                            idx_a.at[pl.ds(0, tail)])
            pltpu.sync_copy(ones_v, deg_sh.at[idx_a], add=True)
        plsc.subcore_barrier()
        pltpu.sync_copy(deg_sh.at[pl.ds(r0, rpt)],
                        out_hbm.at[pl.ds(c * n_acc + r0, rpt)])

    return deg_k


def _make_agg(n_acc, d, ep, sc_tiling=False, ch=CH):
    ew = ep // NW
    cpt = ew // ch
    tail = ew % ch
    rpt = n_acc // NS
    NB = 4
    mesh = plsc.VectorSubcoreMesh(**_MESH)

    @functools.partial(
        pl.kernel,
        mesh=mesh,
        compiler_params=_SC_TILING if sc_tiling else None,
        out_type=jax.ShapeDtypeStruct((NC * n_acc, d), jnp.float32),
        scratch_types=(
            [pltpu.VMEM((ch,), jnp.int32)] * (2 * NB)
            + [pltpu.VMEM((ch, d), jnp.float32)] * NB
            + [pltpu.VMEM_SHARED((n_acc, d), jnp.float32)]
            + [pltpu.SemaphoreType.DMA] * NB
        ),
    )
    def agg_k(u_hbm, src_hbm, dst_hbm, out_hbm, *refs):
        idx_s = refs[0:NB]
        idx_d = refs[NB:2 * NB]
        rows = refs[2 * NB:3 * NB]
        acc_sh = refs[3 * NB]
        sems = refs[3 * NB + 1:4 * NB + 1]
        c = lax.axis_index("c")
        s = lax.axis_index("s")
        wid = c * NS + s
        r0 = s * rpt
        _zero_fill_2d(rows[0], ch, d)
        for k in range(rpt // ch):
            pltpu.sync_copy(rows[0], acc_sh.at[pl.ds(r0 + k * ch, ch)])
        if rpt % ch:
            pltpu.sync_copy(rows[0].at[pl.ds(0, rpt % ch)],
                            acc_sh.at[pl.ds(r0 + (rpt // ch) * ch, rpt % ch)])
        plsc.subcore_barrier()

        def start(b, base):
            pltpu.sync_copy(src_hbm.at[pl.ds(base, ch)], idx_s[b])
            pltpu.sync_copy(dst_hbm.at[pl.ds(base, ch)], idx_d[b])
            return pltpu.async_copy(u_hbm.at[idx_s[b]], rows[b], sems[b])

        def drain(b, cp):
            cp.wait()
            pltpu.sync_copy(rows[b], acc_sh.at[idx_d[b]], add=True)

        def body(i, carry):
            base0 = wid * ew + (NB * i) * ch
            cps = [start(b, base0 + b * ch) for b in range(NB)]
            for b in range(NB):
                drain(b, cps[b])
            return carry

        lax.fori_loop(0, cpt // NB, body, 0)
        rem = cpt % NB
        if rem:
            base0 = wid * ew + (cpt - rem) * ch
            cps = [start(b, base0 + b * ch) for b in range(rem)]
            for b in range(rem):
                drain(b, cps[b])
        if tail:
            base = wid * ew + cpt * ch
            for j in range(ch // 16):
                garb = 16 * j + jnp.arange(16, dtype=jnp.int32)
                idx_s[0][pl.ds(16 * j, 16)] = garb
                idx_d[0][pl.ds(16 * j, 16)] = (n_acc - GARB) + garb % GARB
            pltpu.sync_copy(src_hbm.at[pl.ds(base, tail)],
                            idx_s[0].at[pl.ds(0, tail)])
            pltpu.sync_copy(dst_hbm.at[pl.ds(base, tail)],
                            idx_d[0].at[pl.ds(0, tail)])
            pltpu.async_copy(u_hbm.at[idx_s[0]], rows[0], sems[0]).wait()
            pltpu.sync_copy(rows[0], acc_sh.at[idx_d[0]], add=True)
        plsc.subcore_barrier()
        pltpu.sync_copy(acc_sh.at[pl.ds(r0, rpt)],
                        out_hbm.at[pl.ds(c * n_acc + r0, rpt)])

    return agg_k


def _make_dec(d, ep):
    ew = ep // NW
    cpt = ew // CH
    tail = ew % CH
    mesh = plsc.VectorSubcoreMesh(**_MESH)

    NB = 4

    @functools.partial(
        pl.kernel,
        mesh=mesh,
        compiler_params=_SC_TILING,
        out_type=jax.ShapeDtypeStruct((ep * 16,), jnp.float32),
        scratch_types=(
            [pltpu.VMEM((CH,), jnp.int32)] * (2 * NB)
            + [pltpu.VMEM((CH, d), jnp.float32)] * (2 * NB)
            + [pltpu.VMEM((CH * 16,), jnp.float32)]
            + [pltpu.SemaphoreType.DMA] * (2 * NB)
        ),
    )
    def dec_k(z_hbm, a_hbm, b_hbm, out_hbm, *refs):
        idx_a = refs[0:NB]
        idx_b = refs[NB:2 * NB]
        za = refs[2 * NB:3 * NB]
        zb = refs[3 * NB:4 * NB]
        part = refs[4 * NB]
        sa = refs[4 * NB + 1:5 * NB + 1]
        sb = refs[5 * NB + 1:6 * NB + 1]
        c = lax.axis_index("c")
        s = lax.axis_index("s")
        wid = c * NS + s

        UNR = 4
        assert CH % UNR == 0

        def fill_part(zab, zbb, m):
            def edge(q, carry2):
                e0 = q * UNR
                for u in range(UNR):
                    e2 = e0 + u
                    acc = zab[e2, pl.ds(0, 16)] * zbb[e2, pl.ds(0, 16)]
                    for j in range(1, d // 16):
                        acc = acc + zab[e2, pl.ds(16 * j, 16)] * zbb[e2, pl.ds(16 * j, 16)]
                    part[pl.ds(e2 * 16, 16)] = acc
                return carry2

            lax.fori_loop(0, m // UNR, edge, 0)

        def start(b, base):
            pltpu.sync_copy(a_hbm.at[pl.ds(base, CH)], idx_a[b])
            pltpu.sync_copy(b_hbm.at[pl.ds(base, CH)], idx_b[b])
            return (pltpu.async_copy(z_hbm.at[idx_a[b]], za[b], sa[b]),
                    pltpu.async_copy(z_hbm.at[idx_b[b]], zb[b], sb[b]))

        def drain(b, cp, base):
            cp[0].wait()
            cp[1].wait()
            fill_part(za[b], zb[b], CH)
            pltpu.sync_copy(part, out_hbm.at[pl.ds(base * 16, CH * 16)])

        def body(i, carry):
            base0 = wid * ew + (NB * i) * CH
            cps = [start(b, base0 + b * CH) for b in range(NB)]
            for b in range(NB):
                drain(b, cps[b], base0 + b * CH)
            return carry

        lax.fori_loop(0, cpt // NB, body, 0)
        rem = cpt % NB
        if rem:
            base0 = wid * ew + (cpt - rem) * CH
            cps = [start(b, base0 + b * CH) for b in range(rem)]
            for b in range(rem):
                drain(b, cps[b], base0 + b * CH)
        if tail:
            base = wid * ew + cpt * CH
            for j in range(CH // 16):
                garb = 16 * j + jnp.arange(16, dtype=jnp.int32)
                idx_a[0][pl.ds(16 * j, 16)] = garb
                idx_b[0][pl.ds(16 * j, 16)] = garb
            pltpu.sync_copy(a_hbm.at[pl.ds(base, tail)],
                            idx_a[0].at[pl.ds(0, tail)])
            pltpu.sync_copy(b_hbm.at[pl.ds(base, tail)],
                            idx_b[0].at[pl.ds(0, tail)])
            ca = pltpu.async_copy(z_hbm.at[idx_a[0]], za[0], sa[0])
            cb = pltpu.async_copy(z_hbm.at[idx_b[0]], zb[0], sb[0])
            ca.wait()
            cb.wait()
            fill_part(za[0], zb[0], tail)
            pltpu.sync_copy(part.at[pl.ds(0, tail * 16)],
                            out_hbm.at[pl.ds(base * 16, tail * 16)])

    return dec_k


def _tc_encode1(x, W1, deg_t):
    n, d_hid = x.shape[0], W1.shape[1]

    def body(x_ref, w_ref, deg_ref, u1_ref, dis_ref):
        deg = deg_ref[:, 0:1] + deg_ref[:, 1:2] + 1.0
        dis = lax.rsqrt(deg)
        xw = jnp.dot(x_ref[...], w_ref[...],
                     preferred_element_type=jnp.float32,
                     precision=lax.Precision.HIGHEST)
        u1_ref[...] = xw * dis
        dis_ref[...] = dis

    return pl.pallas_call(
        body,
        out_shape=(jax.ShapeDtypeStruct((n, d_hid), jnp.float32),
                   jax.ShapeDtypeStruct((n, 1), jnp.float32)),
    )(x, W1, deg_t)


def _tc_mid(pa, pb, u1, dis, b1, W2):
    n = u1.shape[0]
    d_out = W2.shape[1]

    def body(pa_ref, pb_ref, u1_ref, dis_ref, b1_ref, w2_ref, u2_ref):
        acc = pa_ref[...] + pb_ref[...] + u1_ref[...]
        h = jnp.maximum(dis_ref[...] * acc + b1_ref[...], 0.0)
        hw = jnp.dot(h, w2_ref[...],
                     preferred_element_type=jnp.float32,
                     precision=lax.Precision.HIGHEST)
        u2_ref[...] = hw * dis_ref[...]

    return pl.pallas_call(
        body,
        out_shape=jax.ShapeDtypeStruct((n, d_out), jnp.float32),
    )(pa, pb, u1, dis, b1, W2)


def _tc_final(pa, pb, u2, dis, b2):
    n, d_out = u2.shape

    def body(pa_ref, pb_ref, u2_ref, dis_ref, b2_ref, z_ref):
        acc = pa_ref[...] + pb_ref[...] + u2_ref[...]
        z_ref[...] = dis_ref[...] * acc + b2_ref[...]

    return pl.pallas_call(
        body,
        out_shape=jax.ShapeDtypeStruct((n, d_out), jnp.float32),
    )(pa, pb, u2, dis, b2)


def _tc_reduce16(p2d, sel):
    m = p2d.shape[0]

    def body(p_ref, s_ref, o_ref):
        o_ref[...] = jnp.dot(p_ref[...], s_ref[...],
                             preferred_element_type=jnp.float32,
                             precision=lax.Precision.HIGHEST)

    return pl.pallas_call(
        body,
        out_shape=jax.ShapeDtypeStruct((m, 128), jnp.float32),
    )(p2d, sel)


@jax.jit
def kernel(x, edge_index, pos_edge_index, neg_edge_index, W1, b1, W2, b2):
    n = x.shape[0]
    d_hid = W1.shape[1]
    d_out = W2.shape[1]
    n_acc = n + GARB

    src, dst = edge_index[0], edge_index[1]
    src_p, dst_p, ep = _pad_edges(src, dst, n)

    ei = jnp.concatenate([pos_edge_index, neg_edge_index], axis=1)
    e_dec = ei.shape[1]
    a_p, b_p, ep_dec = _pad_edges(ei[0], ei[1], n)
    if ep_dec != e_dec:
        b_p = jnp.where(jnp.arange(ep_dec) < e_dec, b_p, b_p % n)

    deg_parts = _make_deg(n_acc, ep)(dst_p).reshape(NC, n_acc)
    deg_t = jnp.transpose(deg_parts[:, :n])

    u1, dis = _tc_encode1(x, W1, deg_t)
    parts1 = _make_agg(n_acc, d_hid, ep, ch=64)(u1, src_p, dst_p)
    u2 = _tc_mid(parts1[:n], parts1[n_acc:n_acc + n], u1, dis,
                 b1.reshape(1, d_hid), W2)

    parts2 = _make_agg(n_acc, d_out, ep, sc_tiling=True)(u2, src_p, dst_p)
    z = _tc_final(parts2[:n], parts2[n_acc:n_acc + n], u2, dis,
                  b2.reshape(1, d_out))

    pf = _make_dec(d_out, ep_dec)(z, a_p, b_p)
    p2d = pf.reshape(ep_dec * 16 // 2048, 2048)
    sel = (jnp.arange(2048, dtype=jnp.int32)[:, None] // 16
           == jnp.arange(128, dtype=jnp.int32)[None, :]).astype(jnp.float32)
    s2 = _tc_reduce16(p2d, sel)
    return s2.reshape(-1)[:e_dec]


# speedup vs baseline: 1.0519x; 1.0519x over previous
import functools

import jax
import jax.numpy as jnp
from jax import lax
from jax.experimental import pallas as pl
from jax.experimental.pallas import tpu as pltpu
from jax.experimental.pallas import tpu_sc as plsc

NC = 2
NS = 16
NW = NC * NS
CH = 128
GARB = 240

_MESH = dict(core_axis_name="c", subcore_axis_name="s")

_SC_TILING = pltpu.CompilerParams(use_tc_tiling_on_sc=False)


def _zero_fill_1d(ref, size):
    def b(i, carry):
        ref[pl.ds(i * 16, 16)] = jnp.zeros((16,), jnp.float32)
        return carry

    lax.fori_loop(0, size // 16, b, 0)


def _zero_fill_2d(ref, r, d):
    def b(i, carry):
        for j in range(d // 16):
            ref[i, pl.ds(16 * j, 16)] = jnp.zeros((16,), jnp.float32)
        return carry

    lax.fori_loop(0, r, b, 0)


def _pad_edges(idx_val, idx_tgt, n):
    e = idx_val.shape[0]
    ep = ((e + NW * 8 - 1) // (NW * 8)) * (NW * 8)
    pad = ep - e
    if pad == 0:
        return idx_val, idx_tgt, ep
    ar = jnp.arange(pad, dtype=jnp.int32)
    val_p = jnp.concatenate([idx_val, ar % n])
    tgt_p = jnp.concatenate([idx_tgt, n + (ar % GARB)])
    return val_p, tgt_p, ep


def _make_deg(n_acc, ep):
    ew = ep // NW
    cpt = ew // CH
    tail = ew % CH
    rpt = n_acc // NS
    mesh = plsc.VectorSubcoreMesh(**_MESH)

    @functools.partial(
        pl.kernel,
        mesh=mesh,
        out_type=jax.ShapeDtypeStruct((NC * n_acc,), jnp.float32),
        scratch_types=[
            pltpu.VMEM((CH,), jnp.int32),
            pltpu.VMEM((CH,), jnp.int32),
            pltpu.VMEM((CH,), jnp.float32),
            pltpu.VMEM((rpt,), jnp.float32),
            pltpu.VMEM_SHARED((n_acc,), jnp.float32),
        ],
    )
    def deg_k(dst_hbm, out_hbm, idx_a, idx_b, ones_v, zbuf, deg_sh):
        c = lax.axis_index("c")
        s = lax.axis_index("s")
        wid = c * NS + s
        for j in range(CH // 16):
            ones_v[pl.ds(16 * j, 16)] = jnp.ones((16,), jnp.float32)
        if CH % 16:
            ones_v[pl.ds(CH - 16, 16)] = jnp.ones((16,), jnp.float32)
        r0 = s * rpt
        _zero_fill_1d(zbuf, rpt)
        pltpu.sync_copy(zbuf, deg_sh.at[pl.ds(r0, rpt)])
        plsc.subcore_barrier()

        def body(i, carry):
            base_a = wid * ew + (2 * i) * CH
            pltpu.sync_copy(dst_hbm.at[pl.ds(base_a, CH)], idx_a)
            pltpu.sync_copy(dst_hbm.at[pl.ds(base_a + CH, CH)], idx_b)
            pltpu.sync_copy(ones_v, deg_sh.at[idx_a], add=True)
            pltpu.sync_copy(ones_v, deg_sh.at[idx_b], add=True)
            return carry

        lax.fori_loop(0, cpt // 2, body, 0)
        if cpt % 2:
            base = wid * ew + (cpt - 1) * CH
            pltpu.sync_copy(dst_hbm.at[pl.ds(base, CH)], idx_a)
            pltpu.sync_copy(ones_v, deg_sh.at[idx_a], add=True)
        if tail:
            base = wid * ew + cpt * CH
            for j in range(CH // 16):
                garb = 16 * j + jnp.arange(16, dtype=jnp.int32)
                idx_a[pl.ds(16 * j, 16)] = (n_acc - GARB) + garb % GARB
            pltpu.sync_copy(dst_hbm.at[pl.ds(base, tail)],
                            idx_a.at[pl.ds(0, tail)])
            pltpu.sync_copy(ones_v, deg_sh.at[idx_a], add=True)
        plsc.subcore_barrier()
        pltpu.sync_copy(deg_sh.at[pl.ds(r0, rpt)],
                        out_hbm.at[pl.ds(c * n_acc + r0, rpt)])

    return deg_k


def _make_agg(n_acc, d, ep, sc_tiling=False, ch=CH, nb=4):
    ew = ep // NW
    cpt = ew // ch
    tail = ew % ch
    rpt = n_acc // NS
    NB = nb
    mesh = plsc.VectorSubcoreMesh(**_MESH)

    @functools.partial(
        pl.kernel,
        mesh=mesh,
        compiler_params=_SC_TILING if sc_tiling else None,
        out_type=jax.ShapeDtypeStruct((NC * n_acc, d), jnp.float32),
        scratch_types=(
            [pltpu.VMEM((ch,), jnp.int32)] * (2 * NB)
            + [pltpu.VMEM((ch, d), jnp.float32)] * NB
            + [pltpu.VMEM_SHARED((n_acc, d), jnp.float32)]
            + [pltpu.SemaphoreType.DMA] * NB
        ),
    )
    def agg_k(u_hbm, src_hbm, dst_hbm, out_hbm, *refs):
        idx_s = refs[0:NB]
        idx_d = refs[NB:2 * NB]
        rows = refs[2 * NB:3 * NB]
        acc_sh = refs[3 * NB]
        sems = refs[3 * NB + 1:4 * NB + 1]
        c = lax.axis_index("c")
        s = lax.axis_index("s")
        wid = c * NS + s
        r0 = s * rpt
        _zero_fill_2d(rows[0], ch, d)
        for k in range(rpt // ch):
            pltpu.sync_copy(rows[0], acc_sh.at[pl.ds(r0 + k * ch, ch)])
        if rpt % ch:
            pltpu.sync_copy(rows[0].at[pl.ds(0, rpt % ch)],
                            acc_sh.at[pl.ds(r0 + (rpt // ch) * ch, rpt % ch)])
        plsc.subcore_barrier()

        def start(b, base):
            pltpu.sync_copy(src_hbm.at[pl.ds(base, ch)], idx_s[b])
            pltpu.sync_copy(dst_hbm.at[pl.ds(base, ch)], idx_d[b])
            return pltpu.async_copy(u_hbm.at[idx_s[b]], rows[b], sems[b])

        def drain(b, cp):
            cp.wait()
            pltpu.sync_copy(rows[b], acc_sh.at[idx_d[b]], add=True)

        def body(i, carry):
            base0 = wid * ew + (NB * i) * ch
            cps = [start(b, base0 + b * ch) for b in range(NB)]
            for b in range(NB):
                drain(b, cps[b])
            return carry

        lax.fori_loop(0, cpt // NB, body, 0)
        rem = cpt % NB
        if rem:
            base0 = wid * ew + (cpt - rem) * ch
            cps = [start(b, base0 + b * ch) for b in range(rem)]
            for b in range(rem):
                drain(b, cps[b])
        if tail:
            base = wid * ew + cpt * ch
            for j in range(ch // 16):
                garb = 16 * j + jnp.arange(16, dtype=jnp.int32)
                idx_s[0][pl.ds(16 * j, 16)] = garb
                idx_d[0][pl.ds(16 * j, 16)] = (n_acc - GARB) + garb % GARB
            pltpu.sync_copy(src_hbm.at[pl.ds(base, tail)],
                            idx_s[0].at[pl.ds(0, tail)])
            pltpu.sync_copy(dst_hbm.at[pl.ds(base, tail)],
                            idx_d[0].at[pl.ds(0, tail)])
            pltpu.async_copy(u_hbm.at[idx_s[0]], rows[0], sems[0]).wait()
            pltpu.sync_copy(rows[0], acc_sh.at[idx_d[0]], add=True)
        plsc.subcore_barrier()
        pltpu.sync_copy(acc_sh.at[pl.ds(r0, rpt)],
                        out_hbm.at[pl.ds(c * n_acc + r0, rpt)])

    return agg_k


def _make_dec(d, ep):
    ew = ep // NW
    cpt = ew // CH
    tail = ew % CH
    mesh = plsc.VectorSubcoreMesh(**_MESH)

    NB = 4

    @functools.partial(
        pl.kernel,
        mesh=mesh,
        compiler_params=_SC_TILING,
        out_type=jax.ShapeDtypeStruct((ep * 16,), jnp.float32),
        scratch_types=(
            [pltpu.VMEM((CH,), jnp.int32)] * (2 * NB)
            + [pltpu.VMEM((CH, d), jnp.float32)] * (2 * NB)
            + [pltpu.VMEM((CH * 16,), jnp.float32)]
            + [pltpu.SemaphoreType.DMA] * (2 * NB)
        ),
    )
    def dec_k(z_hbm, a_hbm, b_hbm, out_hbm, *refs):
        idx_a = refs[0:NB]
        idx_b = refs[NB:2 * NB]
        za = refs[2 * NB:3 * NB]
        zb = refs[3 * NB:4 * NB]
        part = refs[4 * NB]
        sa = refs[4 * NB + 1:5 * NB + 1]
        sb = refs[5 * NB + 1:6 * NB + 1]
        c = lax.axis_index("c")
        s = lax.axis_index("s")
        wid = c * NS + s

        UNR = 4
        assert CH % UNR == 0

        def fill_part(zab, zbb, m):
            def edge(q, carry2):
                e0 = q * UNR
                for u in range(UNR):
                    e2 = e0 + u
                    acc = zab[e2, pl.ds(0, 16)] * zbb[e2, pl.ds(0, 16)]
                    for j in range(1, d // 16):
                        acc = acc + zab[e2, pl.ds(16 * j, 16)] * zbb[e2, pl.ds(16 * j, 16)]
                    part[pl.ds(e2 * 16, 16)] = acc
                return carry2

            lax.fori_loop(0, m // UNR, edge, 0)

        def start(b, base):
            pltpu.sync_copy(a_hbm.at[pl.ds(base, CH)], idx_a[b])
            pltpu.sync_copy(b_hbm.at[pl.ds(base, CH)], idx_b[b])
            return (pltpu.async_copy(z_hbm.at[idx_a[b]], za[b], sa[b]),
                    pltpu.async_copy(z_hbm.at[idx_b[b]], zb[b], sb[b]))

        def drain(b, cp, base):
            cp[0].wait()
            cp[1].wait()
            fill_part(za[b], zb[b], CH)
            pltpu.sync_copy(part, out_hbm.at[pl.ds(base * 16, CH * 16)])

        def body(i, carry):
            base0 = wid * ew + (NB * i) * CH
            cps = [start(b, base0 + b * CH) for b in range(NB)]
            for b in range(NB):
                drain(b, cps[b], base0 + b * CH)
            return carry

        lax.fori_loop(0, cpt // NB, body, 0)
        rem = cpt % NB
        if rem:
            base0 = wid * ew + (cpt - rem) * CH
            cps = [start(b, base0 + b * CH) for b in range(rem)]
            for b in range(rem):
                drain(b, cps[b], base0 + b * CH)
        if tail:
            base = wid * ew + cpt * CH
            for j in range(CH // 16):
                garb = 16 * j + jnp.arange(16, dtype=jnp.int32)
                idx_a[0][pl.ds(16 * j, 16)] = garb
                idx_b[0][pl.ds(16 * j, 16)] = garb
            pltpu.sync_copy(a_hbm.at[pl.ds(base, tail)],
                            idx_a[0].at[pl.ds(0, tail)])
            pltpu.sync_copy(b_hbm.at[pl.ds(base, tail)],
                            idx_b[0].at[pl.ds(0, tail)])
            ca = pltpu.async_copy(z_hbm.at[idx_a[0]], za[0], sa[0])
            cb = pltpu.async_copy(z_hbm.at[idx_b[0]], zb[0], sb[0])
            ca.wait()
            cb.wait()
            fill_part(za[0], zb[0], tail)
            pltpu.sync_copy(part.at[pl.ds(0, tail * 16)],
                            out_hbm.at[pl.ds(base * 16, tail * 16)])

    return dec_k


def _tc_encode1(x, W1, deg_t):
    n, d_hid = x.shape[0], W1.shape[1]

    def body(x_ref, w_ref, deg_ref, u1_ref, dis_ref):
        deg = deg_ref[:, 0:1] + deg_ref[:, 1:2] + 1.0
        dis = lax.rsqrt(deg)
        xw = jnp.dot(x_ref[...], w_ref[...],
                     preferred_element_type=jnp.float32,
                     precision=lax.Precision.HIGHEST)
        u1_ref[...] = xw * dis
        dis_ref[...] = dis

    return pl.pallas_call(
        body,
        out_shape=(jax.ShapeDtypeStruct((n, d_hid), jnp.float32),
                   jax.ShapeDtypeStruct((n, 1), jnp.float32)),
    )(x, W1, deg_t)


def _tc_mid(pa, pb, u1, dis, b1, W2):
    n = u1.shape[0]
    d_out = W2.shape[1]

    def body(pa_ref, pb_ref, u1_ref, dis_ref, b1_ref, w2_ref, u2_ref):
        acc = pa_ref[...] + pb_ref[...] + u1_ref[...]
        h = jnp.maximum(dis_ref[...] * acc + b1_ref[...], 0.0)
        hw = jnp.dot(h, w2_ref[...],
                     preferred_element_type=jnp.float32,
                     precision=lax.Precision.HIGHEST)
        u2_ref[...] = hw * dis_ref[...]

    return pl.pallas_call(
        body,
        out_shape=jax.ShapeDtypeStruct((n, d_out), jnp.float32),
    )(pa, pb, u1, dis, b1, W2)


def _tc_final(pa, pb, u2, dis, b2):
    n, d_out = u2.shape

    def body(pa_ref, pb_ref, u2_ref, dis_ref, b2_ref, z_ref):
        acc = pa_ref[...] + pb_ref[...] + u2_ref[...]
        z_ref[...] = dis_ref[...] * acc + b2_ref[...]

    return pl.pallas_call(
        body,
        out_shape=jax.ShapeDtypeStruct((n, d_out), jnp.float32),
    )(pa, pb, u2, dis, b2)


def _tc_reduce16(p2d, sel):
    m = p2d.shape[0]

    def body(p_ref, s_ref, o_ref):
        o_ref[...] = jnp.dot(p_ref[...], s_ref[...],
                             preferred_element_type=jnp.float32,
                             precision=lax.Precision.HIGHEST)

    return pl.pallas_call(
        body,
        out_shape=jax.ShapeDtypeStruct((m, 128), jnp.float32),
    )(p2d, sel)


@jax.jit
def kernel(x, edge_index, pos_edge_index, neg_edge_index, W1, b1, W2, b2):
    n = x.shape[0]
    d_hid = W1.shape[1]
    d_out = W2.shape[1]
    n_acc = n + GARB

    src, dst = edge_index[0], edge_index[1]
    src_p, dst_p, ep = _pad_edges(src, dst, n)

    ei = jnp.concatenate([pos_edge_index, neg_edge_index], axis=1)
    e_dec = ei.shape[1]
    a_p, b_p, ep_dec = _pad_edges(ei[0], ei[1], n)
    if ep_dec != e_dec:
        b_p = jnp.where(jnp.arange(ep_dec) < e_dec, b_p, b_p % n)

    deg_parts = _make_deg(n_acc, ep)(dst_p).reshape(NC, n_acc)
    deg_t = jnp.transpose(deg_parts[:, :n])

    u1, dis = _tc_encode1(x, W1, deg_t)
    parts1 = _make_agg(n_acc, d_hid, ep, nb=2)(u1, src_p, dst_p)
    u2 = _tc_mid(parts1[:n], parts1[n_acc:n_acc + n], u1, dis,
                 b1.reshape(1, d_hid), W2)

    parts2 = _make_agg(n_acc, d_out, ep, sc_tiling=True)(u2, src_p, dst_p)
    z = _tc_final(parts2[:n], parts2[n_acc:n_acc + n], u2, dis,
                  b2.reshape(1, d_out))

    pf = _make_dec(d_out, ep_dec)(z, a_p, b_p)
    p2d = pf.reshape(ep_dec * 16 // 2048, 2048)
    sel = (jnp.arange(2048, dtype=jnp.int32)[:, None] // 16
           == jnp.arange(128, dtype=jnp.int32)[None, :]).astype(jnp.float32)
    s2 = _tc_reduce16(p2d, sel)
    return s2.reshape(-1)[:e_dec]


# speedup vs baseline: 1.0540x; 1.0020x over previous
import functools

import jax
import jax.numpy as jnp
from jax import lax
from jax.experimental import pallas as pl
from jax.experimental.pallas import tpu as pltpu
from jax.experimental.pallas import tpu_sc as plsc

NC = 2
NS = 16
NW = NC * NS
CH = 128
GARB = 240

_MESH = dict(core_axis_name="c", subcore_axis_name="s")

_SC_TILING = pltpu.CompilerParams(use_tc_tiling_on_sc=False)


def _zero_fill_1d(ref, size):
    def b(i, carry):
        ref[pl.ds(i * 16, 16)] = jnp.zeros((16,), jnp.float32)
        return carry

    lax.fori_loop(0, size // 16, b, 0)


def _zero_fill_2d(ref, r, d):
    def b(i, carry):
        for j in range(d // 16):
            ref[i, pl.ds(16 * j, 16)] = jnp.zeros((16,), jnp.float32)
        return carry

    lax.fori_loop(0, r, b, 0)


def _pad_edges(idx_val, idx_tgt, n):
    e = idx_val.shape[0]
    ep = ((e + NW * 8 - 1) // (NW * 8)) * (NW * 8)
    pad = ep - e
    if pad == 0:
        return idx_val, idx_tgt, ep
    ar = jnp.arange(pad, dtype=jnp.int32)
    val_p = jnp.concatenate([idx_val, ar % n])
    tgt_p = jnp.concatenate([idx_tgt, n + (ar % GARB)])
    return val_p, tgt_p, ep


def _make_deg(n_acc, ep):
    ew = ep // NW
    cpt = ew // CH
    tail = ew % CH
    rpt = n_acc // NS
    mesh = plsc.VectorSubcoreMesh(**_MESH)

    @functools.partial(
        pl.kernel,
        mesh=mesh,
        out_type=jax.ShapeDtypeStruct((NC * n_acc,), jnp.float32),
        scratch_types=[
            pltpu.VMEM((CH,), jnp.int32),
            pltpu.VMEM((CH,), jnp.int32),
            pltpu.VMEM((CH,), jnp.float32),
            pltpu.VMEM((rpt,), jnp.float32),
            pltpu.VMEM_SHARED((n_acc,), jnp.float32),
        ],
    )
    def deg_k(dst_hbm, out_hbm, idx_a, idx_b, ones_v, zbuf, deg_sh):
        c = lax.axis_index("c")
        s = lax.axis_index("s")
        wid = c * NS + s
        for j in range(CH // 16):
            ones_v[pl.ds(16 * j, 16)] = jnp.ones((16,), jnp.float32)
        if CH % 16:
            ones_v[pl.ds(CH - 16, 16)] = jnp.ones((16,), jnp.float32)
        r0 = s * rpt
        _zero_fill_1d(zbuf, rpt)
        pltpu.sync_copy(zbuf, deg_sh.at[pl.ds(r0, rpt)])
        plsc.subcore_barrier()

        def body(i, carry):
            base_a = wid * ew + (2 * i) * CH
            pltpu.sync_copy(dst_hbm.at[pl.ds(base_a, CH)], idx_a)
            pltpu.sync_copy(dst_hbm.at[pl.ds(base_a + CH, CH)], idx_b)
            pltpu.sync_copy(ones_v, deg_sh.at[idx_a], add=True)
            pltpu.sync_copy(ones_v, deg_sh.at[idx_b], add=True)
            return carry

        lax.fori_loop(0, cpt // 2, body, 0)
        if cpt % 2:
            base = wid * ew + (cpt - 1) * CH
            pltpu.sync_copy(dst_hbm.at[pl.ds(base, CH)], idx_a)
            pltpu.sync_copy(ones_v, deg_sh.at[idx_a], add=True)
        if tail:
            base = wid * ew + cpt * CH
            for j in range(CH // 16):
                garb = 16 * j + jnp.arange(16, dtype=jnp.int32)
                idx_a[pl.ds(16 * j, 16)] = (n_acc - GARB) + garb % GARB
            pltpu.sync_copy(dst_hbm.at[pl.ds(base, tail)],
                            idx_a.at[pl.ds(0, tail)])
            pltpu.sync_copy(ones_v, deg_sh.at[idx_a], add=True)
        plsc.subcore_barrier()
        pltpu.sync_copy(deg_sh.at[pl.ds(r0, rpt)],
                        out_hbm.at[pl.ds(c * n_acc + r0, rpt)])

    return deg_k


def _make_agg(n_acc, d, ep, sc_tiling=False, ch=CH, nb=4):
    ew = ep // NW
    cpt = ew // ch
    tail = ew % ch
    rpt = n_acc // NS
    NB = nb
    mesh = plsc.VectorSubcoreMesh(**_MESH)

    @functools.partial(
        pl.kernel,
        mesh=mesh,
        compiler_params=_SC_TILING if sc_tiling else None,
        out_type=jax.ShapeDtypeStruct((NC * n_acc, d), jnp.float32),
        scratch_types=(
            [pltpu.VMEM((ch,), jnp.int32)] * (2 * NB)
            + [pltpu.VMEM((ch, d), jnp.float32)] * NB
            + [pltpu.VMEM_SHARED((n_acc, d), jnp.float32)]
            + [pltpu.SemaphoreType.DMA] * NB
        ),
    )
    def agg_k(u_hbm, src_hbm, dst_hbm, out_hbm, *refs):
        idx_s = refs[0:NB]
        idx_d = refs[NB:2 * NB]
        rows = refs[2 * NB:3 * NB]
        acc_sh = refs[3 * NB]
        sems = refs[3 * NB + 1:4 * NB + 1]
        c = lax.axis_index("c")
        s = lax.axis_index("s")
        wid = c * NS + s
        r0 = s * rpt
        _zero_fill_2d(rows[0], ch, d)
        for k in range(rpt // ch):
            pltpu.sync_copy(rows[0], acc_sh.at[pl.ds(r0 + k * ch, ch)])
        if rpt % ch:
            pltpu.sync_copy(rows[0].at[pl.ds(0, rpt % ch)],
                            acc_sh.at[pl.ds(r0 + (rpt // ch) * ch, rpt % ch)])
        plsc.subcore_barrier()

        def start(b, base):
            pltpu.sync_copy(src_hbm.at[pl.ds(base, ch)], idx_s[b])
            pltpu.sync_copy(dst_hbm.at[pl.ds(base, ch)], idx_d[b])
            return pltpu.async_copy(u_hbm.at[idx_s[b]], rows[b], sems[b])

        def drain(b, cp):
            cp.wait()
            pltpu.sync_copy(rows[b], acc_sh.at[idx_d[b]], add=True)

        def body(i, carry):
            base0 = wid * ew + (NB * i) * ch
            cps = [start(b, base0 + b * ch) for b in range(NB)]
            for b in range(NB):
                drain(b, cps[b])
            return carry

        lax.fori_loop(0, cpt // NB, body, 0)
        rem = cpt % NB
        if rem:
            base0 = wid * ew + (cpt - rem) * ch
            cps = [start(b, base0 + b * ch) for b in range(rem)]
            for b in range(rem):
                drain(b, cps[b])
        if tail:
            base = wid * ew + cpt * ch
            for j in range(ch // 16):
                garb = 16 * j + jnp.arange(16, dtype=jnp.int32)
                idx_s[0][pl.ds(16 * j, 16)] = garb
                idx_d[0][pl.ds(16 * j, 16)] = (n_acc - GARB) + garb % GARB
            pltpu.sync_copy(src_hbm.at[pl.ds(base, tail)],
                            idx_s[0].at[pl.ds(0, tail)])
            pltpu.sync_copy(dst_hbm.at[pl.ds(base, tail)],
                            idx_d[0].at[pl.ds(0, tail)])
            pltpu.async_copy(u_hbm.at[idx_s[0]], rows[0], sems[0]).wait()
            pltpu.sync_copy(rows[0], acc_sh.at[idx_d[0]], add=True)
        plsc.subcore_barrier()
        pltpu.sync_copy(acc_sh.at[pl.ds(r0, rpt)],
                        out_hbm.at[pl.ds(c * n_acc + r0, rpt)])

    return agg_k


def _make_dec(d, ep):
    ew = ep // NW
    cpt = ew // CH
    tail = ew % CH
    mesh = plsc.VectorSubcoreMesh(**_MESH)

    NB = 4

    @functools.partial(
        pl.kernel,
        mesh=mesh,
        compiler_params=_SC_TILING,
        out_type=jax.ShapeDtypeStruct((ep * 16,), jnp.float32),
        scratch_types=(
            [pltpu.VMEM((CH,), jnp.int32)] * (2 * NB)
            + [pltpu.VMEM((CH, d), jnp.float32)] * (2 * NB)
            + [pltpu.VMEM((CH * 16,), jnp.float32)]
            + [pltpu.SemaphoreType.DMA] * (2 * NB)
        ),
    )
    def dec_k(z_hbm, a_hbm, b_hbm, out_hbm, *refs):
        idx_a = refs[0:NB]
        idx_b = refs[NB:2 * NB]
        za = refs[2 * NB:3 * NB]
        zb = refs[3 * NB:4 * NB]
        part = refs[4 * NB]
        sa = refs[4 * NB + 1:5 * NB + 1]
        sb = refs[5 * NB + 1:6 * NB + 1]
        c = lax.axis_index("c")
        s = lax.axis_index("s")
        wid = c * NS + s

        UNR = 8
        assert CH % UNR == 0

        def fill_part(zab, zbb, m):
            def edge(q, carry2):
                e0 = q * UNR
                for u in range(UNR):
                    e2 = e0 + u
                    acc = zab[e2, pl.ds(0, 16)] * zbb[e2, pl.ds(0, 16)]
                    for j in range(1, d // 16):
                        acc = acc + zab[e2, pl.ds(16 * j, 16)] * zbb[e2, pl.ds(16 * j, 16)]
                    part[pl.ds(e2 * 16, 16)] = acc
                return carry2

            lax.fori_loop(0, m // UNR, edge, 0)

        def start(b, base):
            pltpu.sync_copy(a_hbm.at[pl.ds(base, CH)], idx_a[b])
            pltpu.sync_copy(b_hbm.at[pl.ds(base, CH)], idx_b[b])
            return (pltpu.async_copy(z_hbm.at[idx_a[b]], za[b], sa[b]),
                    pltpu.async_copy(z_hbm.at[idx_b[b]], zb[b], sb[b]))

        def drain(b, cp, base):
            cp[0].wait()
            cp[1].wait()
            fill_part(za[b], zb[b], CH)
            pltpu.sync_copy(part, out_hbm.at[pl.ds(base * 16, CH * 16)])

        def body(i, carry):
            base0 = wid * ew + (NB * i) * CH
            cps = [start(b, base0 + b * CH) for b in range(NB)]
            for b in range(NB):
                drain(b, cps[b], base0 + b * CH)
            return carry

        lax.fori_loop(0, cpt // NB, body, 0)
        rem = cpt % NB
        if rem:
            base0 = wid * ew + (cpt - rem) * CH
            cps = [start(b, base0 + b * CH) for b in range(rem)]
            for b in range(rem):
                drain(b, cps[b], base0 + b * CH)
        if tail:
            base = wid * ew + cpt * CH
            for j in range(CH // 16):
                garb = 16 * j + jnp.arange(16, dtype=jnp.int32)
                idx_a[0][pl.ds(16 * j, 16)] = garb
                idx_b[0][pl.ds(16 * j, 16)] = garb
            pltpu.sync_copy(a_hbm.at[pl.ds(base, tail)],
                            idx_a[0].at[pl.ds(0, tail)])
            pltpu.sync_copy(b_hbm.at[pl.ds(base, tail)],
                            idx_b[0].at[pl.ds(0, tail)])
            ca = pltpu.async_copy(z_hbm.at[idx_a[0]], za[0], sa[0])
            cb = pltpu.async_copy(z_hbm.at[idx_b[0]], zb[0], sb[0])
            ca.wait()
            cb.wait()
            fill_part(za[0], zb[0], tail)
            pltpu.sync_copy(part.at[pl.ds(0, tail * 16)],
                            out_hbm.at[pl.ds(base * 16, tail * 16)])

    return dec_k


def _tc_encode1(x, W1, deg_t):
    n, d_hid = x.shape[0], W1.shape[1]

    def body(x_ref, w_ref, deg_ref, u1_ref, dis_ref):
        deg = deg_ref[:, 0:1] + deg_ref[:, 1:2] + 1.0
        dis = lax.rsqrt(deg)
        xw = jnp.dot(x_ref[...], w_ref[...],
                     preferred_element_type=jnp.float32,
                     precision=lax.Precision.HIGHEST)
        u1_ref[...] = xw * dis
        dis_ref[...] = dis

    return pl.pallas_call(
        body,
        out_shape=(jax.ShapeDtypeStruct((n, d_hid), jnp.float32),
                   jax.ShapeDtypeStruct((n, 1), jnp.float32)),
    )(x, W1, deg_t)


def _tc_mid(pa, pb, u1, dis, b1, W2):
    n = u1.shape[0]
    d_out = W2.shape[1]

    def body(pa_ref, pb_ref, u1_ref, dis_ref, b1_ref, w2_ref, u2_ref):
        acc = pa_ref[...] + pb_ref[...] + u1_ref[...]
        h = jnp.maximum(dis_ref[...] * acc + b1_ref[...], 0.0)
        hw = jnp.dot(h, w2_ref[...],
                     preferred_element_type=jnp.float32,
                     precision=lax.Precision.HIGHEST)
        u2_ref[...] = hw * dis_ref[...]

    return pl.pallas_call(
        body,
        out_shape=jax.ShapeDtypeStruct((n, d_out), jnp.float32),
    )(pa, pb, u1, dis, b1, W2)


def _tc_final(pa, pb, u2, dis, b2):
    n, d_out = u2.shape

    def body(pa_ref, pb_ref, u2_ref, dis_ref, b2_ref, z_ref):
        acc = pa_ref[...] + pb_ref[...] + u2_ref[...]
        z_ref[...] = dis_ref[...] * acc + b2_ref[...]

    return pl.pallas_call(
        body,
        out_shape=jax.ShapeDtypeStruct((n, d_out), jnp.float32),
    )(pa, pb, u2, dis, b2)


def _tc_reduce16(p2d, sel):
    m = p2d.shape[0]

    def body(p_ref, s_ref, o_ref):
        o_ref[...] = jnp.dot(p_ref[...], s_ref[...],
                             preferred_element_type=jnp.float32,
                             precision=lax.Precision.HIGHEST)

    return pl.pallas_call(
        body,
        out_shape=jax.ShapeDtypeStruct((m, 128), jnp.float32),
    )(p2d, sel)


@jax.jit
def kernel(x, edge_index, pos_edge_index, neg_edge_index, W1, b1, W2, b2):
    n = x.shape[0]
    d_hid = W1.shape[1]
    d_out = W2.shape[1]
    n_acc = n + GARB

    src, dst = edge_index[0], edge_index[1]
    src_p, dst_p, ep = _pad_edges(src, dst, n)

    ei = jnp.concatenate([pos_edge_index, neg_edge_index], axis=1)
    e_dec = ei.shape[1]
    a_p, b_p, ep_dec = _pad_edges(ei[0], ei[1], n)
    if ep_dec != e_dec:
        b_p = jnp.where(jnp.arange(ep_dec) < e_dec, b_p, b_p % n)

    deg_parts = _make_deg(n_acc, ep)(dst_p).reshape(NC, n_acc)
    deg_t = jnp.transpose(deg_parts[:, :n])

    u1, dis = _tc_encode1(x, W1, deg_t)
    parts1 = _make_agg(n_acc, d_hid, ep, nb=2)(u1, src_p, dst_p)
    u2 = _tc_mid(parts1[:n], parts1[n_acc:n_acc + n], u1, dis,
                 b1.reshape(1, d_hid), W2)

    parts2 = _make_agg(n_acc, d_out, ep, sc_tiling=True)(u2, src_p, dst_p)
    z = _tc_final(parts2[:n], parts2[n_acc:n_acc + n], u2, dis,
                  b2.reshape(1, d_out))

    pf = _make_dec(d_out, ep_dec)(z, a_p, b_p)
    p2d = pf.reshape(ep_dec * 16 // 2048, 2048)
    sel = (jnp.arange(2048, dtype=jnp.int32)[:, None] // 16
           == jnp.arange(128, dtype=jnp.int32)[None, :]).astype(jnp.float32)
    s2 = _tc_reduce16(p2d, sel)
    return s2.reshape(-1)[:e_dec]
